# baseline (speedup 1.0000x reference)
import jax
import jax.numpy as jnp
from jax import lax
from jax.experimental import pallas as pl
from jax.experimental.pallas import tpu as pltpu
from jax.experimental.pallas import tpu_sc as plsc

N_FIELDS = 26
VOCAB = 100000
EMB_DIM = 16
BATCH = 16384

NC, NS, LANES = 2, 16, 16
NW = NC * NS
ROWS = BATCH * N_FIELDS
R_PER_W = ROWS // NW
CHUNK = 3328
N_CHUNKS = R_PER_W // CHUNK
STEPS = CHUNK // LANES


def _body(xflat, offs, table, out, idx_v, rows_v, offs_v, sem):
    wid = lax.axis_index("s") * NC + lax.axis_index("c")
    base0 = wid * R_PER_W
    pltpu.sync_copy(offs, offs_v)
    for c in range(N_CHUNKS):
        base = base0 + c * CHUNK
        pltpu.sync_copy(xflat.at[pl.ds(base, CHUNK)], idx_v)

        def step(j, carry):
            sl = pl.ds(j * LANES, LANES)
            idx_v[sl] = idx_v[sl] + offs_v[sl]
            return carry

        lax.fori_loop(0, STEPS, step, 0)
        pltpu.async_copy(table.at[idx_v], rows_v, sem).wait()
        pltpu.sync_copy(rows_v, out.at[pl.ds(base, CHUNK)])


def kernel(x_cat, W):
    xflat = x_cat.reshape(ROWS).astype(jnp.int32)
    table = W.reshape(N_FIELDS * VOCAB, EMB_DIM)
    offs = (jnp.arange(CHUNK, dtype=jnp.int32) % N_FIELDS) * VOCAB
    k = pl.kernel(
        _body,
        mesh=plsc.VectorSubcoreMesh(core_axis_name="c", subcore_axis_name="s"),
        compiler_params=pltpu.CompilerParams(use_tc_tiling_on_sc=False),
        out_type=jax.ShapeDtypeStruct((ROWS, EMB_DIM), jnp.float32),
        scratch_types=[
            pltpu.VMEM((CHUNK,), jnp.int32),
            pltpu.VMEM((CHUNK, EMB_DIM), jnp.float32),
            pltpu.VMEM((CHUNK,), jnp.int32),
            pltpu.SemaphoreType.DMA,
        ],
    )
    out = k(xflat, offs, table)
    return out.reshape(BATCH, N_FIELDS, EMB_DIM)


# speedup vs baseline: 1.2135x; 1.2135x over previous
import jax
import jax.numpy as jnp
from jax import lax
from jax.experimental import pallas as pl
from jax.experimental.pallas import tpu as pltpu
from jax.experimental.pallas import tpu_sc as plsc

N_FIELDS = 26
VOCAB = 100000
EMB_DIM = 16
BATCH = 16384

NC, NS, LANES = 2, 16, 16
NW = NC * NS
B_PER_W = BATCH // NW


def _body(xt, W, out, idx_v, rows_v, sem):
    wid = lax.axis_index("s") * NC + lax.axis_index("c")
    b0 = wid * B_PER_W

    def per_field(f, carry):
        pltpu.sync_copy(xt.at[f, pl.ds(b0, B_PER_W)], idx_v)
        pltpu.async_copy(W.at[f].at[idx_v], rows_v, sem).wait()
        pltpu.sync_copy(rows_v, out.at[f, pl.ds(b0, B_PER_W)])
        return carry

    lax.fori_loop(0, N_FIELDS, per_field, 0)


def kernel(x_cat, W):
    xt = x_cat.T
    k = pl.kernel(
        _body,
        mesh=plsc.VectorSubcoreMesh(core_axis_name="c", subcore_axis_name="s"),
        out_type=jax.ShapeDtypeStruct((N_FIELDS, BATCH, EMB_DIM), jnp.float32),
        scratch_types=[
            pltpu.VMEM((B_PER_W,), jnp.int32),
            pltpu.VMEM((B_PER_W, EMB_DIM), jnp.float32),
            pltpu.SemaphoreType.DMA,
        ],
        compiler_params=pltpu.CompilerParams(use_tc_tiling_on_sc=False),
    )
    out = k(xt, W)
    return out.transpose(1, 0, 2)


# speedup vs baseline: 1.2444x; 1.0255x over previous
import jax
import jax.numpy as jnp
from jax import lax
from jax.experimental import pallas as pl
from jax.experimental.pallas import tpu as pltpu
from jax.experimental.pallas import tpu_sc as plsc

N_FIELDS = 26
VOCAB = 100000
EMB_DIM = 16
BATCH = 16384

NC, NS = 2, 16
NW = NC * NS
B_PER_W = BATCH // NW
PAIRS = N_FIELDS // 2


def _body(xt, W, out, idx_v, rows_a, rows_b, sem_a, sem_b):
    wid = lax.axis_index("s") * NC + lax.axis_index("c")
    b0 = wid * B_PER_W

    pltpu.sync_copy(xt.at[:, pl.ds(b0, B_PER_W)], idx_v)

    def gather(f, rows, sem):
        pltpu.make_async_copy(W.at[f].at[idx_v.at[f]], rows, sem).start()

    def drain(f, rows, sem):
        pltpu.make_async_copy(W.at[f].at[idx_v.at[f]], rows, sem).wait()

    gather(0, rows_a, sem_a)

    def per_pair(g, carry):
        f0 = 2 * g
        gather(f0 + 1, rows_b, sem_b)
        drain(f0, rows_a, sem_a)
        pltpu.sync_copy(rows_a, out.at[f0, pl.ds(b0, B_PER_W)])

        @pl.when(g < PAIRS - 1)
        def _():
            gather(f0 + 2, rows_a, sem_a)

        drain(f0 + 1, rows_b, sem_b)
        pltpu.sync_copy(rows_b, out.at[f0 + 1, pl.ds(b0, B_PER_W)])
        return carry

    lax.fori_loop(0, PAIRS, per_pair, 0)


def kernel(x_cat, W):
    xt = x_cat.T
    k = pl.kernel(
        _body,
        mesh=plsc.VectorSubcoreMesh(core_axis_name="c", subcore_axis_name="s"),
        out_type=jax.ShapeDtypeStruct((N_FIELDS, BATCH, EMB_DIM), jnp.float32),
        scratch_types=[
            pltpu.VMEM((N_FIELDS, B_PER_W), jnp.int32),
            pltpu.VMEM((B_PER_W, EMB_DIM), jnp.float32),
            pltpu.VMEM((B_PER_W, EMB_DIM), jnp.float32),
            pltpu.SemaphoreType.DMA,
            pltpu.SemaphoreType.DMA,
        ],
        compiler_params=pltpu.CompilerParams(use_tc_tiling_on_sc=False),
    )
    out = k(xt, W)
    return out.transpose(1, 0, 2)
